# baseline (speedup 1.0000x reference)
import functools

import jax
import jax.numpy as jnp
from jax import lax
from jax.experimental import pallas as pl
from jax.experimental.pallas import tpu as pltpu
from jax.experimental.pallas import tpu_sc as plsc

_B = 1024
_S = 50
_H = 128
_T = _B * _S
_BLK = 800
_BB = 16
_GRID = _T // _BLK

_NC = 2
_NS = 16
_NW = _NC * _NS
_PER_W = _T // _NW
_CH = 80
_NCH = _PER_W // _CH


_BPW = _B // _NW


def _sc_gather(item_table, url_table, item_idx, url_idx):
  mesh = plsc.VectorSubcoreMesh(core_axis_name="c", subcore_axis_name="s")

  @functools.partial(
      pl.kernel,
      out_type=[
          jax.ShapeDtypeStruct((_T, 64), jnp.float32),
          jax.ShapeDtypeStruct((_T, 64), jnp.float32),
      ],
      mesh=mesh,
      compiler_params=pltpu.CompilerParams(use_tc_tiling_on_sc=False),
      scratch_types=[
          pltpu.VMEM((_BPW, _S), jnp.int32),
          pltpu.VMEM((_PER_W, 64), jnp.float32),
          pltpu.SemaphoreType.DMA,
      ],
  )
  def k(item_t, url_t, iidx, uidx, item_out, url_out, idx_v, rows_v, sem):
    wid = lax.axis_index("s") * _NC + lax.axis_index("c")
    base = wid * _PER_W
    for tab, idx_hbm, out in ((item_t, iidx, item_out), (url_t, uidx, url_out)):
      pltpu.sync_copy(idx_hbm.at[pl.ds(wid * _BPW, _BPW)], idx_v)
      copies = []
      for r in range(_BPW):
        copies.append(
            pltpu.async_copy(
                tab.at[idx_v.at[r]],
                rows_v.at[pl.ds(r * _S, _S)], sem))
      for c in copies:
        c.wait()
      pltpu.sync_copy(rows_v, out.at[pl.ds(base, _PER_W)])

  return k(item_table, url_table, item_idx, url_idx)


def _ln(x, g, b):
  d = x.shape[-1]
  jm = jnp.full((d, d), 1.0 / d, jnp.float32)
  m = jnp.dot(x, jm, preferred_element_type=jnp.float32)
  xc = x - m
  v = jnp.dot(xc * xc, jm, preferred_element_type=jnp.float32)
  inv = lax.rsqrt(v + 1e-5)
  return xc * inv * g + b


def _dot(a, b):
  return jnp.dot(a, b, preferred_element_type=jnp.float32)


def _onehot(ids_col, width):
  i = lax.broadcasted_iota(jnp.int32, (ids_col.shape[0], width), 1)
  return (ids_col == i).astype(jnp.float32)


def _tc_body(ev_ref, cat_ref, pr_ref, nm_ref, qr_ref, it_ref, ur_ref,
             evt_ref, ctt_ref, prt_ref, wdt_ref,
             ev_g, ev_b, ct_g, ct_b, pr_g, pr_b, wd_g, wd_b,
             it_g, it_b, ur_g, ur_b,
             ipW, ipb, ip_g, ip_b,
             upW, upb, up_g, up_b,
             nmW, nmb, nm_g, nm_b,
             iW1, ib1, i_g, i_b, iW2, ib2,
             fW1, fb1, f1g, f1b, fW2, fb2, f2g, f2b,
             out_ref):
  wdt = wdt_ref[...]

  def bag(ids3):
    ids = ids3.reshape(_BLK, 16)
    t = ids.shape[0]
    iw = lax.broadcasted_iota(jnp.int16, (t, 384), 1)
    counts = jnp.zeros((t, 384), jnp.int16)
    for w in range(16):
      idw = ids[:, w:w + 1]
      idw16 = jnp.where(idw == 0, -1, idw).astype(jnp.int16)
      counts += (idw16 == iw).astype(jnp.int16)
    countsf = counts.astype(jnp.float32)
    cnt = _dot(countsf, jnp.full((384, 1), 1.0, jnp.float32))
    s = _dot(countsf, wdt)
    return jnp.where(cnt > 0, s / jnp.maximum(cnt, 1.0), 0.0)

  event_emb = _ln(_dot(_onehot(ev_ref[...], 8), evt_ref[...]), ev_g[...], ev_b[...])
  cat_emb = _ln(_dot(_onehot(cat_ref[...], 1024), ctt_ref[...]), ct_g[...], ct_b[...])
  price_emb = _ln(_dot(_onehot(pr_ref[...], 128), prt_ref[...]), pr_g[...], pr_b[...])

  name_emb = _ln(bag(nm_ref[...]), wd_g[...], wd_b[...])
  name_emb = _ln(jax.nn.relu(_dot(name_emb, nmW[...]) + nmb[...]), nm_g[...], nm_b[...])
  query_emb = _ln(bag(qr_ref[...]), wd_g[...], wd_b[...])

  item_emb = _ln(it_ref[...], it_g[...], it_b[...])
  item_emb = jax.nn.relu(_ln(_dot(item_emb, ipW[...]) + ipb[...], ip_g[...], ip_b[...]))
  url_emb = _ln(ur_ref[...], ur_g[...], ur_b[...])
  url_emb = jax.nn.relu(_ln(_dot(url_emb, upW[...]) + upb[...], up_g[...], up_b[...]))

  feats = [event_emb, cat_emb, price_emb, name_emb, query_emb, item_emb, url_emb]

  j128 = jnp.full((_H, _H), 1.0, jnp.float32)
  fstack = jnp.concatenate(feats, axis=0)
  n2 = _dot(fstack * fstack, j128)
  nstack = fstack * lax.rsqrt(jnp.maximum(n2, 1e-24))
  h = jax.nn.relu(_ln(_dot(nstack, iW1[...]) + ib1[...], i_g[...], i_b[...]))
  s = jax.nn.sigmoid(_dot(h, iW2[...]) + ib2[...])

  t = ev_ref.shape[0]
  scores = [s[i * t:(i + 1) * t] for i in range(7)]
  m = scores[0]
  for sc in scores[1:]:
    m = jnp.maximum(m, sc)
  es = [jnp.exp(sc - m) for sc in scores]
  z = es[0]
  for e in es[1:]:
    z = z + e
  ws = [e / z for e in es]

  cf = jnp.concatenate([f * w for f, w in zip(feats, ws)], axis=1)
  h1 = jax.nn.relu(_ln(_dot(cf, fW1[...]) + fb1[...], f1g[...], f1b[...]))
  out = jnp.tanh(_ln(_dot(h1, fW2[...]) + fb2[...], f2g[...], f2b[...]))
  out = jnp.where(jnp.isnan(out), jnp.zeros_like(out), out)
  out_ref[...] = jnp.clip(out, -5.0, 5.0).reshape(_BB, _S, _H)


def _tc_forward(ev, cat, pr, nm, qr, item_rows, url_rows, weights):
  def tok2(shape):
    return pl.BlockSpec((_BLK,) + shape[1:], lambda i: (i,) + (0,) * (len(shape) - 1))

  def full(a):
    return pl.BlockSpec(a.shape, lambda i: (0,) * a.ndim)

  bat3 = pl.BlockSpec((_BB, _S, 16), lambda i: (i, 0, 0))
  args = [ev, cat, pr, nm, qr, item_rows, url_rows] + weights
  in_specs = ([tok2(a.shape) for a in [ev, cat, pr]] + [bat3, bat3] +
              [tok2(item_rows.shape), tok2(url_rows.shape)] +
              [full(w) for w in weights])

  return pl.pallas_call(
      _tc_body,
      grid=(_GRID,),
      in_specs=in_specs,
      out_specs=pl.BlockSpec((_BB, _S, _H), lambda i: (i, 0, 0)),
      out_shape=jax.ShapeDtypeStruct((_B, _S, _H), jnp.float32),
  )(*args)


def kernel(event_types, categories, prices, names, queries, timestamps,
           item_ids, urls, params):
  p = params
  ev = event_types.reshape(_T, 1).astype(jnp.int32)
  cat = categories.reshape(_T, 1).astype(jnp.int32)
  pr = prices.reshape(_T, 1).astype(jnp.int32)
  nm = names.astype(jnp.int32)
  qr = queries.astype(jnp.int32)
  iidx = item_ids.astype(jnp.int32)
  uidx = urls.astype(jnp.int32)

  item_rows, url_rows = _sc_gather(p['item_table'], p['url_table'], iidx, uidx)

  def row2(a):
    return a.reshape(1, -1)

  weights = [
      jnp.pad(p['event_table'], ((0, 1), (0, 0))),
      jnp.pad(p['cat_table'], ((0, 24), (0, 0))),
      jnp.pad(p['price_table'], ((0, 26), (0, 0))),
      jnp.pad(p['word_table'], ((0, 125), (0, 0))),
      row2(p['event_ln_g']), row2(p['event_ln_b']),
      row2(p['cat_ln_g']), row2(p['cat_ln_b']),
      row2(p['price_ln_g']), row2(p['price_ln_b']),
      row2(p['word_ln_g']), row2(p['word_ln_b']),
      row2(p['item_ln_g']), row2(p['item_ln_b']),
      row2(p['url_ln_g']), row2(p['url_ln_b']),
      p['item_proj_W'], row2(p['item_proj_b']),
      row2(p['item_proj_ln_g']), row2(p['item_proj_ln_b']),
      p['url_proj_W'], row2(p['url_proj_b']),
      row2(p['url_proj_ln_g']), row2(p['url_proj_ln_b']),
      p['nm_W'], row2(p['nm_b']), row2(p['nm_ln_g']), row2(p['nm_ln_b']),
      p['imp_W1'], row2(p['imp_b1']), row2(p['imp_ln_g']), row2(p['imp_ln_b']),
      p['imp_W2'], row2(p['imp_b2']),
      p['fus_W1'], row2(p['fus_b1']), row2(p['fus_ln1_g']), row2(p['fus_ln1_b']),
      p['fus_W2'], row2(p['fus_b2']), row2(p['fus_ln2_g']), row2(p['fus_ln2_b']),
  ]

  out = _tc_forward(ev, cat, pr, nm, qr, item_rows, url_rows, weights)
  return out


# speedup vs baseline: 1.0342x; 1.0342x over previous
import functools

import jax
import jax.numpy as jnp
from jax import lax
from jax.experimental import pallas as pl
from jax.experimental.pallas import tpu as pltpu
from jax.experimental.pallas import tpu_sc as plsc

_B = 1024
_S = 50
_H = 128
_T = _B * _S
_BLK = 1600
_BB = 32
_GRID = _T // _BLK

_NC = 2
_NS = 16
_NW = _NC * _NS
_PER_W = _T // _NW
_CH = 80
_NCH = _PER_W // _CH


_BPW = _B // _NW


def _sc_gather(item_table, url_table, item_idx, url_idx):
  mesh = plsc.VectorSubcoreMesh(core_axis_name="c", subcore_axis_name="s")

  @functools.partial(
      pl.kernel,
      out_type=[
          jax.ShapeDtypeStruct((_T, 64), jnp.float32),
          jax.ShapeDtypeStruct((_T, 64), jnp.float32),
      ],
      mesh=mesh,
      compiler_params=pltpu.CompilerParams(use_tc_tiling_on_sc=False),
      scratch_types=[
          pltpu.VMEM((_BPW, _S), jnp.int32),
          pltpu.VMEM((_PER_W, 64), jnp.float32),
          pltpu.SemaphoreType.DMA,
      ],
  )
  def k(item_t, url_t, iidx, uidx, item_out, url_out, idx_v, rows_v, sem):
    wid = lax.axis_index("s") * _NC + lax.axis_index("c")
    base = wid * _PER_W
    for tab, idx_hbm, out in ((item_t, iidx, item_out), (url_t, uidx, url_out)):
      pltpu.sync_copy(idx_hbm.at[pl.ds(wid * _BPW, _BPW)], idx_v)
      copies = []
      for r in range(_BPW):
        copies.append(
            pltpu.async_copy(
                tab.at[idx_v.at[r]],
                rows_v.at[pl.ds(r * _S, _S)], sem))
      for c in copies:
        c.wait()
      pltpu.sync_copy(rows_v, out.at[pl.ds(base, _PER_W)])

  return k(item_table, url_table, item_idx, url_idx)


def _ln(x, g, b):
  d = x.shape[-1]
  jm = jnp.full((d, d), 1.0 / d, jnp.float32)
  m = jnp.dot(x, jm, preferred_element_type=jnp.float32)
  xc = x - m
  v = jnp.dot(xc * xc, jm, preferred_element_type=jnp.float32)
  inv = lax.rsqrt(v + 1e-5)
  return xc * inv * g + b


def _dot(a, b):
  return jnp.dot(a, b, preferred_element_type=jnp.float32)


def _onehot(ids_col, width):
  i = lax.broadcasted_iota(jnp.int32, (ids_col.shape[0], width), 1)
  return (ids_col == i).astype(jnp.float32)


def _tc_body(ev_ref, cat_ref, pr_ref, nm_ref, qr_ref, it_ref, ur_ref,
             evt_ref, ctt_ref, prt_ref, wdt_ref,
             ev_g, ev_b, ct_g, ct_b, pr_g, pr_b, wd_g, wd_b,
             it_g, it_b, ur_g, ur_b,
             ipW, ipb, ip_g, ip_b,
             upW, upb, up_g, up_b,
             nmW, nmb, nm_g, nm_b,
             iW1, ib1, i_g, i_b, iW2, ib2,
             fW1, fb1, f1g, f1b, fW2, fb2, f2g, f2b,
             out_ref):
  wdt = wdt_ref[...]

  def bag(ids3):
    ids = ids3.reshape(_BLK, 16)
    t = ids.shape[0]
    iw = lax.broadcasted_iota(jnp.int16, (t, 384), 1)
    counts = jnp.zeros((t, 384), jnp.int16)
    for w in range(16):
      idw = ids[:, w:w + 1]
      idw16 = jnp.where(idw == 0, -1, idw).astype(jnp.int16)
      counts += (idw16 == iw).astype(jnp.int16)
    countsf = counts.astype(jnp.float32)
    cnt = _dot(countsf, jnp.full((384, 1), 1.0, jnp.float32))
    s = _dot(countsf, wdt)
    return jnp.where(cnt > 0, s / jnp.maximum(cnt, 1.0), 0.0)

  event_emb = _ln(_dot(_onehot(ev_ref[...], 8), evt_ref[...]), ev_g[...], ev_b[...])
  cat_emb = _ln(_dot(_onehot(cat_ref[...], 1024), ctt_ref[...]), ct_g[...], ct_b[...])
  price_emb = _ln(_dot(_onehot(pr_ref[...], 128), prt_ref[...]), pr_g[...], pr_b[...])

  name_emb = _ln(bag(nm_ref[...]), wd_g[...], wd_b[...])
  name_emb = _ln(jax.nn.relu(_dot(name_emb, nmW[...]) + nmb[...]), nm_g[...], nm_b[...])
  query_emb = _ln(bag(qr_ref[...]), wd_g[...], wd_b[...])

  item_emb = _ln(it_ref[...], it_g[...], it_b[...])
  item_emb = jax.nn.relu(_ln(_dot(item_emb, ipW[...]) + ipb[...], ip_g[...], ip_b[...]))
  url_emb = _ln(ur_ref[...], ur_g[...], ur_b[...])
  url_emb = jax.nn.relu(_ln(_dot(url_emb, upW[...]) + upb[...], up_g[...], up_b[...]))

  feats = [event_emb, cat_emb, price_emb, name_emb, query_emb, item_emb, url_emb]

  j128 = jnp.full((_H, _H), 1.0, jnp.float32)
  fstack = jnp.concatenate(feats, axis=0)
  n2 = _dot(fstack * fstack, j128)
  nstack = fstack * lax.rsqrt(jnp.maximum(n2, 1e-24))
  h = jax.nn.relu(_ln(_dot(nstack, iW1[...]) + ib1[...], i_g[...], i_b[...]))
  s = jax.nn.sigmoid(_dot(h, iW2[...]) + ib2[...])

  t = ev_ref.shape[0]
  scores = [s[i * t:(i + 1) * t] for i in range(7)]
  m = scores[0]
  for sc in scores[1:]:
    m = jnp.maximum(m, sc)
  es = [jnp.exp(sc - m) for sc in scores]
  z = es[0]
  for e in es[1:]:
    z = z + e
  ws = [e / z for e in es]

  cf = jnp.concatenate([f * w for f, w in zip(feats, ws)], axis=1)
  h1 = jax.nn.relu(_ln(_dot(cf, fW1[...]) + fb1[...], f1g[...], f1b[...]))
  out = jnp.tanh(_ln(_dot(h1, fW2[...]) + fb2[...], f2g[...], f2b[...]))
  out = jnp.where(jnp.isnan(out), jnp.zeros_like(out), out)
  out_ref[...] = jnp.clip(out, -5.0, 5.0).reshape(_BB, _S, _H)


def _tc_forward(ev, cat, pr, nm, qr, item_rows, url_rows, weights):
  def tok2(shape):
    return pl.BlockSpec((_BLK,) + shape[1:], lambda i: (i,) + (0,) * (len(shape) - 1))

  def full(a):
    return pl.BlockSpec(a.shape, lambda i: (0,) * a.ndim)

  bat3 = pl.BlockSpec((_BB, _S, 16), lambda i: (i, 0, 0))
  args = [ev, cat, pr, nm, qr, item_rows, url_rows] + weights
  in_specs = ([tok2(a.shape) for a in [ev, cat, pr]] + [bat3, bat3] +
              [tok2(item_rows.shape), tok2(url_rows.shape)] +
              [full(w) for w in weights])

  return pl.pallas_call(
      _tc_body,
      grid=(_GRID,),
      in_specs=in_specs,
      out_specs=pl.BlockSpec((_BB, _S, _H), lambda i: (i, 0, 0)),
      out_shape=jax.ShapeDtypeStruct((_B, _S, _H), jnp.float32),
  )(*args)


def kernel(event_types, categories, prices, names, queries, timestamps,
           item_ids, urls, params):
  p = params
  ev = event_types.reshape(_T, 1).astype(jnp.int32)
  cat = categories.reshape(_T, 1).astype(jnp.int32)
  pr = prices.reshape(_T, 1).astype(jnp.int32)
  nm = names.astype(jnp.int32)
  qr = queries.astype(jnp.int32)
  iidx = item_ids.astype(jnp.int32)
  uidx = urls.astype(jnp.int32)

  item_rows, url_rows = _sc_gather(p['item_table'], p['url_table'], iidx, uidx)

  def row2(a):
    return a.reshape(1, -1)

  weights = [
      jnp.pad(p['event_table'], ((0, 1), (0, 0))),
      jnp.pad(p['cat_table'], ((0, 24), (0, 0))),
      jnp.pad(p['price_table'], ((0, 26), (0, 0))),
      jnp.pad(p['word_table'], ((0, 125), (0, 0))),
      row2(p['event_ln_g']), row2(p['event_ln_b']),
      row2(p['cat_ln_g']), row2(p['cat_ln_b']),
      row2(p['price_ln_g']), row2(p['price_ln_b']),
      row2(p['word_ln_g']), row2(p['word_ln_b']),
      row2(p['item_ln_g']), row2(p['item_ln_b']),
      row2(p['url_ln_g']), row2(p['url_ln_b']),
      p['item_proj_W'], row2(p['item_proj_b']),
      row2(p['item_proj_ln_g']), row2(p['item_proj_ln_b']),
      p['url_proj_W'], row2(p['url_proj_b']),
      row2(p['url_proj_ln_g']), row2(p['url_proj_ln_b']),
      p['nm_W'], row2(p['nm_b']), row2(p['nm_ln_g']), row2(p['nm_ln_b']),
      p['imp_W1'], row2(p['imp_b1']), row2(p['imp_ln_g']), row2(p['imp_ln_b']),
      p['imp_W2'], row2(p['imp_b2']),
      p['fus_W1'], row2(p['fus_b1']), row2(p['fus_ln1_g']), row2(p['fus_ln1_b']),
      p['fus_W2'], row2(p['fus_b2']), row2(p['fus_ln2_g']), row2(p['fus_ln2_b']),
  ]

  out = _tc_forward(ev, cat, pr, nm, qr, item_rows, url_rows, weights)
  return out
